# Initial kernel scaffold; baseline (speedup 1.0000x reference)
import jax
import jax.numpy as jnp
from jax.experimental import pallas as pl


def kernel(boxes, scores):
    raise NotImplementedError("write your pallas kernel here")


# speedup vs baseline: 26.3836x; 26.3836x over previous
import jax
import jax.numpy as jnp
from jax import lax
from jax.experimental import pallas as pl

IOU_THRESHOLD = 0.5
SCORE_THRESHOLD = 0.05
MAX_DETECTIONS = 300
N_BOXES = 20000

ROWS = 160
COLS = 128
NEG = -1e9
VALID_MIN = -1e8
BIG_I = 1 << 30


def _nms_body(y1_ref, x1_ref, y2_ref, x2_ref, s_ref, out_ref):
    y1 = y1_ref[...]
    x1 = x1_ref[...]
    y2 = y2_ref[...]
    x2 = x2_ref[...]
    area = (y2 - y1) * (x2 - x1)

    s0 = s_ref[...]
    s_init = jnp.where(s0 > SCORE_THRESHOLD, s0, NEG)

    out_ref[...] = jnp.zeros_like(out_ref)

    rid = lax.broadcasted_iota(jnp.int32, (ROWS, COLS), 0)
    cid = lax.broadcasted_iota(jnp.int32, (ROWS, COLS), 1)
    lin = rid * COLS + cid
    lane = lax.broadcasted_iota(jnp.int32, (1, COLS), 1)

    def body(i, s):
        m = jnp.max(s)
        valid = m > VALID_MIN
        p = jnp.min(jnp.where(s == m, lin, BIG_I))
        r = p // COLS
        c = p - r * COLS
        lane_oh = (lane == c).astype(jnp.float32)
        by1 = jnp.sum(y1_ref[pl.ds(r, 1), :] * lane_oh)
        bx1 = jnp.sum(x1_ref[pl.ds(r, 1), :] * lane_oh)
        by2 = jnp.sum(y2_ref[pl.ds(r, 1), :] * lane_oh)
        bx2 = jnp.sum(x2_ref[pl.ds(r, 1), :] * lane_oh)

        yy1 = jnp.maximum(by1, y1)
        xx1 = jnp.maximum(bx1, x1)
        yy2 = jnp.minimum(by2, y2)
        xx2 = jnp.minimum(bx2, x2)
        inter = jnp.maximum(yy2 - yy1, 0.0) * jnp.maximum(xx2 - xx1, 0.0)
        area_b = (by2 - by1) * (bx2 - bx1)
        iou = inter / jnp.maximum(area_b + area - inter, 1e-9)
        s = jnp.where(valid & (iou > IOU_THRESHOLD), NEG, s)

        vf = jnp.float32(valid)
        row = (
            jnp.where(lane == 0, by1, 0.0)
            + jnp.where(lane == 1, bx1, 0.0)
            + jnp.where(lane == 2, by2, 0.0)
            + jnp.where(lane == 3, bx2, 0.0)
            + jnp.where(lane == 4, m, 0.0)
        ) * vf
        out_ref[pl.ds(i, 1), :] = row
        return s

    lax.fori_loop(0, MAX_DETECTIONS, body, s_init, unroll=False)


def kernel(boxes, scores):
    pad = ROWS * COLS - N_BOXES
    y1 = jnp.pad(boxes[:, 0], (0, pad)).reshape(ROWS, COLS)
    x1 = jnp.pad(boxes[:, 1], (0, pad)).reshape(ROWS, COLS)
    y2 = jnp.pad(boxes[:, 2], (0, pad)).reshape(ROWS, COLS)
    x2 = jnp.pad(boxes[:, 3], (0, pad)).reshape(ROWS, COLS)
    s = jnp.pad(scores, (0, pad)).reshape(ROWS, COLS)

    out = pl.pallas_call(
        _nms_body,
        out_shape=jax.ShapeDtypeStruct((304, COLS), jnp.float32),
    )(y1, x1, y2, x2, s)
    return out[:MAX_DETECTIONS, :5]
